# Initial kernel scaffold; baseline (speedup 1.0000x reference)
import jax
import jax.numpy as jnp
from jax.experimental import pallas as pl


def kernel(x, edge_index, d0_index, Qw, Qb, Kw, Kb):
    raise NotImplementedError("write your pallas kernel here")


# speedup vs baseline: 16.8828x; 16.8828x over previous
import functools

import jax
import jax.numpy as jnp
from jax import lax
from jax.experimental import pallas as pl
from jax.experimental.pallas import tpu as pltpu
from jax.experimental.pallas import tpu_sc as plsc

N = 10000
E = 320000
D = 128
AD = 64
T = 2 * AD

NC = 2
NS = 16
NW = NC * NS
L = 16

EPW = E // NW
C = 80
NCHUNK = EPW // C
NP2 = 10240
SW = NP2 // NS


def _proj_body(x_ref, qw_ref, kw_ref, qb_ref, kb_ref, o_ref):
    xb = x_ref[...]
    dn = (((1,), (1,)), ((), ()))
    q = lax.dot_general(xb, qw_ref[...], dn, preferred_element_type=jnp.float32)
    k = lax.dot_general(xb, kw_ref[...], dn, preferred_element_type=jnp.float32)
    o_ref[...] = jnp.concatenate([q + qb_ref[...], k + kb_ref[...]], axis=1)


def _project(x, Qw, Qb, Kw, Kb):
    R = 1024
    NP = 10240
    xp = jnp.pad(x, ((0, NP - N), (0, 0)))
    out = pl.pallas_call(
        _proj_body,
        grid=(NP // R,),
        in_specs=[
            pl.BlockSpec((R, D), lambda i: (i, 0)),
            pl.BlockSpec((AD, D), lambda i: (0, 0)),
            pl.BlockSpec((AD, D), lambda i: (0, 0)),
            pl.BlockSpec((1, AD), lambda i: (0, 0)),
            pl.BlockSpec((1, AD), lambda i: (0, 0)),
        ],
        out_specs=pl.BlockSpec((R, T), lambda i: (i, 0)),
        out_shape=jax.ShapeDtypeStruct((NP, T), jnp.float32),
    )(xp, Qw, Kw, Qb.reshape(1, AD), Kb.reshape(1, AD))
    return out[:N]


def _sc_body(t_hbm, src_hbm, dst_hbm, d0_hbm, zeros_hbm,
             a1_hbm, a0_hbm,
             sidx, didx, rows_s, rows_d, d0b, scores, hist,
             red_in, red_acc, shared, sem_s, sem_d):
    c = lax.axis_index("c")
    s = lax.axis_index("s")
    wid = s * NC + c
    base = wid * EPW

    pltpu.sync_copy(zeros_hbm, hist)

    iot = lax.iota(jnp.int32, L)

    def chunk_body(i, carry):
        cb = base + i * C
        pltpu.sync_copy(src_hbm.at[pl.ds(cb, C)], sidx)
        pltpu.sync_copy(dst_hbm.at[pl.ds(cb, C)], didx)
        cp_s = pltpu.async_copy(t_hbm.at[sidx], rows_s, sem_s)
        cp_d = pltpu.async_copy(t_hbm.at[didx], rows_d, sem_d)
        pltpu.sync_copy(d0_hbm.at[pl.ds(2 * cb, 2 * C)], d0b)
        cp_s.wait()
        cp_d.wait()

        for g in range(C // L):

            def ebody(eo, svec):
                e = g * L + eo
                acc = jnp.zeros((L,), jnp.float32)
                for j in range(T // L):
                    sv = rows_s[e, pl.ds(j * L, L)]
                    dv = rows_d[e, pl.ds(((j + 4) % 8) * L, L)]
                    acc = acc + sv * dv
                return jnp.where(iot == eo, jnp.sum(acc), svec)

            pre = lax.fori_loop(0, L, ebody, jnp.zeros((L,), jnp.float32),
                                unroll=4)
            v = jnp.exp(0.125 * pre)
            scores[pl.ds(g * L, L)] = v
            le = (iot + g * L) * 2
            ie = plsc.load_gather(d0b, [le])
            io = plsc.load_gather(d0b, [le + 1])
            plsc.addupdate_scatter(hist, [ie], v)
            plsc.addupdate_scatter(hist, [io], v)
        pltpu.sync_copy(scores, a1_hbm.at[pl.ds(cb, C)])
        return carry

    lax.fori_loop(0, NCHUNK, chunk_body, 0)

    off = s * SW
    pltpu.sync_copy(hist, shared.at[s])
    plsc.subcore_barrier()
    pltpu.sync_copy(shared.at[0, pl.ds(off, SW)], red_acc)

    def rbody(r, carry):
        pltpu.sync_copy(shared.at[r, pl.ds(off, SW)], red_in)

        def vbody(jv, carry2):
            sl = pl.ds(jv * L, L)
            red_acc[sl] = red_acc[sl] + red_in[sl]
            return carry2

        lax.fori_loop(0, SW // L, vbody, 0, unroll=8)
        return carry

    lax.fori_loop(1, NS, rbody, 0)
    pltpu.sync_copy(red_acc, a0_hbm.at[c, pl.ds(off, SW)])


@functools.lru_cache(maxsize=None)
def _build_sc_edge():
    return pl.kernel(
        _sc_body,
        out_type=[
            jax.ShapeDtypeStruct((E,), jnp.float32),
            jax.ShapeDtypeStruct((NC, NP2), jnp.float32),
        ],
        mesh=plsc.VectorSubcoreMesh(core_axis_name="c", subcore_axis_name="s",
                                    num_cores=NC, num_subcores=NS),
        scratch_types=[
            pltpu.VMEM((C,), jnp.int32),
            pltpu.VMEM((C,), jnp.int32),
            pltpu.VMEM((C, T), jnp.float32),
            pltpu.VMEM((C, T), jnp.float32),
            pltpu.VMEM((2 * C,), jnp.int32),
            pltpu.VMEM((C,), jnp.float32),
            pltpu.VMEM((NP2,), jnp.float32),
            pltpu.VMEM((SW,), jnp.float32),
            pltpu.VMEM((SW,), jnp.float32),
            pltpu.VMEM_SHARED((NS, NP2), jnp.float32),
            pltpu.SemaphoreType.DMA,
            pltpu.SemaphoreType.DMA,
        ],
        compiler_params=pltpu.CompilerParams(needs_layout_passes=False),
    )


def kernel(x, edge_index, d0_index, Qw, Qb, Kw, Kb):
    t = _project(x, Qw, Qb, Kw, Kb)
    src = edge_index[0]
    dst = edge_index[1]
    d0 = d0_index[1]
    zeros = jnp.zeros((NP2,), jnp.float32)
    diagA1, a0_part = _build_sc_edge()(t, src, dst, d0, zeros)
    diagA0 = a0_part[:, :N].sum(axis=0)
    return (diagA0, diagA1)


# speedup vs baseline: 29.6513x; 1.7563x over previous
import functools

import jax
import jax.numpy as jnp
from jax import lax
from jax.experimental import pallas as pl
from jax.experimental.pallas import tpu as pltpu
from jax.experimental.pallas import tpu_sc as plsc

N = 10000
E = 320000
D = 128
AD = 64
T = 2 * AD

NC = 2
NS = 16
NW = NC * NS
L = 16

EPW = E // NW
C = 80
NCHUNK = EPW // C
NP2 = 10240
SW = NP2 // NS


def _proj_body(x_ref, qw_ref, kw_ref, qb_ref, kb_ref, o_ref):
    xb = x_ref[...]
    dn = (((1,), (1,)), ((), ()))
    q = lax.dot_general(xb, qw_ref[...], dn, preferred_element_type=jnp.float32)
    k = lax.dot_general(xb, kw_ref[...], dn, preferred_element_type=jnp.float32)
    o_ref[...] = jnp.concatenate([q + qb_ref[...], k + kb_ref[...]], axis=1)


def _project(x, Qw, Qb, Kw, Kb):
    R = 1024
    NP = 10240
    xp = jnp.pad(x, ((0, NP - N), (0, 0)))
    out = pl.pallas_call(
        _proj_body,
        grid=(NP // R,),
        in_specs=[
            pl.BlockSpec((R, D), lambda i: (i, 0)),
            pl.BlockSpec((AD, D), lambda i: (0, 0)),
            pl.BlockSpec((AD, D), lambda i: (0, 0)),
            pl.BlockSpec((1, AD), lambda i: (0, 0)),
            pl.BlockSpec((1, AD), lambda i: (0, 0)),
        ],
        out_specs=pl.BlockSpec((R, T), lambda i: (i, 0)),
        out_shape=jax.ShapeDtypeStruct((NP, T), jnp.float32),
    )(xp, Qw, Kw, Qb.reshape(1, AD), Kb.reshape(1, AD))
    return out[:N]


def _sc_body(t_hbm, src_hbm, dst_hbm, d0_hbm, zeros_hbm,
             a1_hbm, a0_hbm,
             sidx0, sidx1, didx0, didx1, d0ba, d0bb,
             rows_s0, rows_d0, rows_s1, rows_d1,
             scores, hist, red_in, red_acc, shared,
             sem_g0, sem_g1, sem_i0, sem_i1):
    c = lax.axis_index("c")
    s = lax.axis_index("s")
    wid = s * NC + c
    base = wid * EPW

    rows = ((rows_s0, rows_d0), (rows_s1, rows_d1))
    sidx = (sidx0, sidx1)
    didx = (didx0, didx1)
    d0b2 = (d0ba, d0bb)
    sem_g = (sem_g0, sem_g1)
    sem_i = (sem_i0, sem_i1)

    pltpu.sync_copy(zeros_hbm, hist)

    iot = lax.iota(jnp.int32, L)

    def idx_srcs(i):
        cb = base + i * C
        return (src_hbm.at[pl.ds(cb, C)], dst_hbm.at[pl.ds(cb, C)],
                d0_hbm.at[pl.ds(2 * cb, 2 * C)])

    def issue_idx(i, p):
        a, b, d = idx_srcs(i)
        pltpu.async_copy(a, sidx[p], sem_i[p])
        pltpu.async_copy(b, didx[p], sem_i[p])
        pltpu.async_copy(d, d0b2[p], sem_i[p])

    def wait_idx(i, p):
        a, b, d = idx_srcs(i)
        pltpu.make_async_copy(a, sidx[p], sem_i[p]).wait()
        pltpu.make_async_copy(b, didx[p], sem_i[p]).wait()
        pltpu.make_async_copy(d, d0b2[p], sem_i[p]).wait()

    def issue_gather(p):
        pltpu.async_copy(t_hbm.at[sidx[p]], rows[p][0], sem_g[p])
        pltpu.async_copy(t_hbm.at[didx[p]], rows[p][1], sem_g[p])

    def wait_gather(p):
        pltpu.make_async_copy(t_hbm.at[sidx[p]], rows[p][0],
                              sem_g[p]).wait()
        pltpu.make_async_copy(t_hbm.at[didx[p]], rows[p][1],
                              sem_g[p]).wait()

    def compute(i, p):
        rows_s, rows_d = rows[p]
        d0b = d0b2[p]
        cb = base + i * C
        for g in range(C // L):

            def ebody(eo, svec):
                e = g * L + eo
                acc = jnp.zeros((L,), jnp.float32)
                for j in range(T // L):
                    sv = rows_s[e, pl.ds(j * L, L)]
                    dv = rows_d[e, pl.ds(((j + 4) % 8) * L, L)]
                    acc = acc + sv * dv
                return jnp.where(iot == eo, jnp.sum(acc), svec)

            pre = lax.fori_loop(0, L, ebody, jnp.zeros((L,), jnp.float32),
                                unroll=4)
            v = jnp.exp(0.125 * pre)
            scores[pl.ds(g * L, L)] = v
            le = (iot + g * L) * 2
            ie = plsc.load_gather(d0b, [le])
            io = plsc.load_gather(d0b, [le + 1])
            plsc.addupdate_scatter(hist, [ie], v)
            plsc.addupdate_scatter(hist, [io], v)
        pltpu.sync_copy(scores, a1_hbm.at[pl.ds(cb, C)])

    issue_idx(0, 0)
    wait_idx(0, 0)
    issue_gather(0)
    issue_idx(1, 1)

    def kbody(k, carry):
        for p in (0, 1):
            i = 2 * k + p
            q = 1 - p

            @pl.when(i < NCHUNK)
            def _do_chunk():
                wait_gather(p)

                @pl.when(i + 1 < NCHUNK)
                def _next_gather():
                    wait_idx(i + 1, q)
                    issue_gather(q)

                @pl.when(i + 2 < NCHUNK)
                def _next_idx():
                    issue_idx(i + 2, p)

                compute(i, p)
        return carry

    lax.fori_loop(0, (NCHUNK + 1) // 2, kbody, 0)

    off = s * SW
    pltpu.sync_copy(hist, shared.at[s])
    plsc.subcore_barrier()
    pltpu.sync_copy(shared.at[0, pl.ds(off, SW)], red_acc)

    def rbody(r, carry):
        pltpu.sync_copy(shared.at[r, pl.ds(off, SW)], red_in)

        def vbody(jv, carry2):
            sl = pl.ds(jv * L, L)
            red_acc[sl] = red_acc[sl] + red_in[sl]
            return carry2

        lax.fori_loop(0, SW // L, vbody, 0, unroll=8)
        return carry

    lax.fori_loop(1, NS, rbody, 0)
    pltpu.sync_copy(red_acc, a0_hbm.at[c, pl.ds(off, SW)])


@functools.lru_cache(maxsize=None)
def _build_sc_edge():
    return pl.kernel(
        _sc_body,
        out_type=[
            jax.ShapeDtypeStruct((E,), jnp.float32),
            jax.ShapeDtypeStruct((NC, NP2), jnp.float32),
        ],
        mesh=plsc.VectorSubcoreMesh(core_axis_name="c", subcore_axis_name="s",
                                    num_cores=NC, num_subcores=NS),
        scratch_types=[
            pltpu.VMEM((C,), jnp.int32),
            pltpu.VMEM((C,), jnp.int32),
            pltpu.VMEM((C,), jnp.int32),
            pltpu.VMEM((C,), jnp.int32),
            pltpu.VMEM((2 * C,), jnp.int32),
            pltpu.VMEM((2 * C,), jnp.int32),
            pltpu.VMEM((C, T), jnp.float32),
            pltpu.VMEM((C, T), jnp.float32),
            pltpu.VMEM((C, T), jnp.float32),
            pltpu.VMEM((C, T), jnp.float32),
            pltpu.VMEM((C,), jnp.float32),
            pltpu.VMEM((NP2,), jnp.float32),
            pltpu.VMEM((SW,), jnp.float32),
            pltpu.VMEM((SW,), jnp.float32),
            pltpu.VMEM_SHARED((NS, NP2), jnp.float32),
            pltpu.SemaphoreType.DMA,
            pltpu.SemaphoreType.DMA,
            pltpu.SemaphoreType.DMA,
            pltpu.SemaphoreType.DMA,
        ],
        compiler_params=pltpu.CompilerParams(needs_layout_passes=False),
    )


def kernel(x, edge_index, d0_index, Qw, Qb, Kw, Kb):
    t = _project(x, Qw, Qb, Kw, Kb)
    src = edge_index[0]
    dst = edge_index[1]
    d0 = d0_index[1]
    zeros = jnp.zeros((NP2,), jnp.float32)
    diagA1, a0_part = _build_sc_edge()(t, src, dst, d0, zeros)
    diagA0 = a0_part[:, :N].sum(axis=0)
    return (diagA0, diagA1)
